# Initial kernel scaffold; baseline (speedup 1.0000x reference)
import jax
import jax.numpy as jnp
from jax.experimental import pallas as pl


def kernel(U_ids, I_ids, R, user_table, item_table, Theta_u, Psi_i, Pi_u, Lambda_u, Lambda_i, A, Bu, Bi, B):
    raise NotImplementedError("write your pallas kernel here")


# speedup vs baseline: 5.4274x; 5.4274x over previous
import functools

import jax
import jax.numpy as jnp
from jax import lax
from jax.experimental import pallas as pl
from jax.experimental.pallas import tpu as pltpu
from jax.experimental.pallas import tpu_sc as plsc

F32 = jnp.float32

NU = 100000
NI = 100000
NF = 16
NA = 8
NT = 32
BATCH = 16384
TW = NA * NT

L_U = 0.01
L_I = 0.01
L_A = 0.001
L_B = 0.01

NC = 2
NS = 16
NW = NC * NS
BPW = BATCH // NW
CH = 64
NCH = BPW // CH


def _sc_gather_body(uids, iids, ut, it, th, ps, usm, ism,
                    out_th, out_ps, out_u, out_i, out_us, out_is,
                    uidx, iidx, thbuf, psbuf, ubuf, ibuf, usbuf, isbuf,
                    sem_s, sem_t):
    wid = lax.axis_index("s") * NC + lax.axis_index("c")
    base = wid * BPW
    pltpu.sync_copy(uids.at[pl.ds(base, BPW)], uidx)
    pltpu.sync_copy(iids.at[pl.ds(base, BPW)], iidx)

    cps = [pltpu.async_copy(ut.at[uidx], ubuf, sem_s),
           pltpu.async_copy(it.at[iidx], ibuf, sem_s),
           pltpu.async_copy(usm.at[uidx], usbuf, sem_s),
           pltpu.async_copy(ism.at[iidx], isbuf, sem_s)]
    for c in range(NCH):
        b = c % 2
        tcp = pltpu.async_copy(th.at[uidx.at[pl.ds(c * CH, CH)]],
                               thbuf.at[b], sem_t)
        pcp = pltpu.async_copy(ps.at[iidx.at[pl.ds(c * CH, CH)]],
                               psbuf.at[b], sem_t)
        tcp.wait()
        pcp.wait()
        pltpu.sync_copy(thbuf.at[b], out_th.at[pl.ds(base + c * CH, CH)])
        pltpu.sync_copy(psbuf.at[b], out_ps.at[pl.ds(base + c * CH, CH)])
    for cp in cps:
        cp.wait()
    pltpu.sync_copy(ubuf, out_u.at[pl.ds(base, BPW)])
    pltpu.sync_copy(ibuf, out_i.at[pl.ds(base, BPW)])
    pltpu.sync_copy(usbuf, out_us.at[pl.ds(base, BPW)])
    pltpu.sync_copy(isbuf, out_is.at[pl.ds(base, BPW)])


@functools.cache
def _sc_gather():
    return pl.kernel(
        _sc_gather_body,
        out_type=[
            jax.ShapeDtypeStruct((BATCH, TW), F32),
            jax.ShapeDtypeStruct((BATCH, TW), F32),
            jax.ShapeDtypeStruct((BATCH, NF), F32),
            jax.ShapeDtypeStruct((BATCH, NF), F32),
            jax.ShapeDtypeStruct((BATCH, 16), F32),
            jax.ShapeDtypeStruct((BATCH, 16), F32),
        ],
        mesh=plsc.VectorSubcoreMesh(core_axis_name="c", subcore_axis_name="s"),
        compiler_params=pltpu.CompilerParams(use_tc_tiling_on_sc=False),
        scratch_types=[
            pltpu.VMEM((BPW,), jnp.int32),
            pltpu.VMEM((BPW,), jnp.int32),
            pltpu.VMEM((2, CH, TW), F32),
            pltpu.VMEM((2, CH, TW), F32),
            pltpu.VMEM((BPW, NF), F32),
            pltpu.VMEM((BPW, NF), F32),
            pltpu.VMEM((BPW, 16), F32),
            pltpu.VMEM((BPW, 16), F32),
            pltpu.SemaphoreType.DMA,
            pltpu.SemaphoreType.DMA,
        ],
    )


BLK = 1024
NBLK = BATCH // BLK


def _tc_math_body(th_ref, ps_ref, u_ref, i_ref, us_ref, is_ref, r_ref,
                  a_ref, b_ref, rhat_ref, ar_ref, loss_ref):
    p = th_ref[...]
    q = ps_ref[...]
    m = 0.5 * (p + q)
    lm = jnp.log(m)
    t = p * (jnp.log(p) - lm) + q * (jnp.log(q) - lm)
    asp = lax.broadcasted_iota(jnp.int32, (TW, NA), 0) // NT
    e = (asp == lax.broadcasted_iota(jnp.int32, (TW, NA), 1)).astype(F32)
    kl = jnp.dot(t, e, preferred_element_type=F32)
    s_uia = 1.0 - 0.5 * kl

    u = u_ref[...]
    i = i_ref[...]
    a = a_ref[...]
    ar = jnp.dot(u * i, (a * a).T, preferred_element_type=F32)
    a_hat = s_uia * ar
    ar_ref[...] = a_hat

    us = us_ref[...]
    isv = is_ref[...]
    pi = us[:, 0:1]
    bu = us[:, 1]
    lu = us[:, 2:10]
    bi = isv[:, 0]
    li = isv[:, 1:9]
    p_uia = pi * lu + (1.0 - pi) * li
    rhat = jnp.sum(p_uia * a_hat, axis=1) + bu + bi + b_ref[0]
    rhat_ref[...] = rhat

    res = r_ref[...] - rhat
    part = 0.5 * jnp.sum(res * res)
    part += 0.5 * L_U * jnp.sum(u * u)
    part += 0.5 * L_I * jnp.sum(i * i)
    part += 0.5 * L_B * (jnp.sum(bu * bu) + jnp.sum(bi * bi))

    @pl.when(pl.program_id(0) == 0)
    def _init():
        loss_ref[0] = 0.5 * L_A * jnp.sum(jnp.abs(a))

    loss_ref[0] += part


_tc_math = pl.pallas_call(
    _tc_math_body,
    grid=(NBLK,),
    in_specs=[
        pl.BlockSpec((BLK, TW), lambda b: (b, 0)),
        pl.BlockSpec((BLK, TW), lambda b: (b, 0)),
        pl.BlockSpec((BLK, NF), lambda b: (b, 0)),
        pl.BlockSpec((BLK, NF), lambda b: (b, 0)),
        pl.BlockSpec((BLK, 16), lambda b: (b, 0)),
        pl.BlockSpec((BLK, 16), lambda b: (b, 0)),
        pl.BlockSpec((BLK,), lambda b: (b,)),
        pl.BlockSpec((NA, NF), lambda b: (0, 0)),
        pl.BlockSpec((1,), lambda b: (0,)),
    ],
    out_specs=[
        pl.BlockSpec((BLK,), lambda b: (b,)),
        pl.BlockSpec((BLK, NA), lambda b: (b, 0)),
        pl.BlockSpec(memory_space=pltpu.SMEM),
    ],
    out_shape=[
        jax.ShapeDtypeStruct((BATCH,), F32),
        jax.ShapeDtypeStruct((BATCH, NA), F32),
        jax.ShapeDtypeStruct((1,), F32),
    ],
)


def kernel(U_ids, I_ids, R, user_table, item_table, Theta_u, Psi_i, Pi_u,
           Lambda_u, Lambda_i, A, Bu, Bi, B):
    th_flat = Theta_u.reshape(NU, TW)
    ps_flat = Psi_i.reshape(NI, TW)
    zu = jnp.zeros((NU, 6), F32)
    usm = jnp.concatenate([Pi_u[:, None], Bu[:, None], Lambda_u, zu], axis=1)
    zi = jnp.zeros((NI, 7), F32)
    ism = jnp.concatenate([Bi[:, None], Lambda_i, zi], axis=1)

    th_b, ps_b, u_b, i_b, us_b, is_b = _sc_gather()(
        U_ids, I_ids, user_table, item_table, th_flat, ps_flat, usm, ism)

    rhat, a_hat, loss = _tc_math(th_b, ps_b, u_b, i_b, us_b, is_b, R, A, B)
    return rhat, a_hat, loss[0]


# speedup vs baseline: 7.0863x; 1.3057x over previous
import functools

import jax
import jax.numpy as jnp
from jax import lax
from jax.experimental import pallas as pl
from jax.experimental.pallas import tpu as pltpu
from jax.experimental.pallas import tpu_sc as plsc

F32 = jnp.float32

NU = 100000
NI = 100000
NF = 16
NA = 8
NT = 32
BATCH = 16384
TW = NA * NT

L_U = 0.01
L_I = 0.01
L_A = 0.001
L_B = 0.01

NC = 2
NS = 16
NW = NC * NS
BPW = BATCH // NW
CH = 64
NCH = BPW // CH


def _sc_theta_body(uids, iids, th, ps, out_th, out_ps,
                   uidx, iidx, thbuf, psbuf, sg0, sg1, sw0, sw1):
    wid = lax.axis_index("s") * NC + lax.axis_index("c")
    base = wid * BPW
    pltpu.sync_copy(uids.at[pl.ds(base, BPW)], uidx)
    pltpu.sync_copy(iids.at[pl.ds(base, BPW)], iidx)

    gsem = (sg0, sg1)
    wsem = (sw0, sw1)
    gp = [None, None]
    wp = [None, None]

    def issue(c):
        b = c % 2
        if wp[b] is not None:
            for cp in wp[b]:
                cp.wait()
            wp[b] = None
        sl = pl.ds(c * CH, CH)
        gp[b] = [pltpu.async_copy(th.at[uidx.at[sl]], thbuf.at[b], gsem[b]),
                 pltpu.async_copy(ps.at[iidx.at[sl]], psbuf.at[b], gsem[b])]

    issue(0)
    for c in range(NCH):
        b = c % 2
        if c + 1 < NCH:
            issue(c + 1)
        for cp in gp[b]:
            cp.wait()
        osl = pl.ds(base + c * CH, CH)
        wp[b] = [pltpu.async_copy(thbuf.at[b], out_th.at[osl], wsem[b]),
                 pltpu.async_copy(psbuf.at[b], out_ps.at[osl], wsem[b])]
    for b in range(2):
        if wp[b] is not None:
            for cp in wp[b]:
                cp.wait()


@functools.cache
def _sc_theta():
    return pl.kernel(
        _sc_theta_body,
        out_type=[
            jax.ShapeDtypeStruct((BATCH, TW), F32),
            jax.ShapeDtypeStruct((BATCH, TW), F32),
        ],
        mesh=plsc.VectorSubcoreMesh(core_axis_name="c", subcore_axis_name="s"),
        scratch_types=[
            pltpu.VMEM((BPW,), jnp.int32),
            pltpu.VMEM((BPW,), jnp.int32),
            pltpu.VMEM((2, CH, TW), F32),
            pltpu.VMEM((2, CH, TW), F32),
            pltpu.SemaphoreType.DMA,
            pltpu.SemaphoreType.DMA,
            pltpu.SemaphoreType.DMA,
            pltpu.SemaphoreType.DMA,
        ],
    )


def _sc_small_body(uids, iids, ut, it, usm, ism,
                   out_u, out_i, out_us, out_is,
                   uidx, iidx, ubuf, ibuf, usbuf, isbuf, sem):
    wid = lax.axis_index("s") * NC + lax.axis_index("c")
    base = wid * BPW
    pltpu.sync_copy(uids.at[pl.ds(base, BPW)], uidx)
    pltpu.sync_copy(iids.at[pl.ds(base, BPW)], iidx)
    cps = [pltpu.async_copy(ut.at[uidx], ubuf, sem),
           pltpu.async_copy(it.at[iidx], ibuf, sem),
           pltpu.async_copy(usm.at[uidx], usbuf, sem),
           pltpu.async_copy(ism.at[iidx], isbuf, sem)]
    for cp in cps:
        cp.wait()
    osl = pl.ds(base, BPW)
    pltpu.sync_copy(ubuf, out_u.at[osl])
    pltpu.sync_copy(ibuf, out_i.at[osl])
    pltpu.sync_copy(usbuf, out_us.at[osl])
    pltpu.sync_copy(isbuf, out_is.at[osl])


@functools.cache
def _sc_small():
    return pl.kernel(
        _sc_small_body,
        out_type=[
            jax.ShapeDtypeStruct((BATCH, NF), F32),
            jax.ShapeDtypeStruct((BATCH, NF), F32),
            jax.ShapeDtypeStruct((BATCH, 16), F32),
            jax.ShapeDtypeStruct((BATCH, 16), F32),
        ],
        mesh=plsc.VectorSubcoreMesh(core_axis_name="c", subcore_axis_name="s"),
        compiler_params=pltpu.CompilerParams(use_tc_tiling_on_sc=False),
        scratch_types=[
            pltpu.VMEM((BPW,), jnp.int32),
            pltpu.VMEM((BPW,), jnp.int32),
            pltpu.VMEM((BPW, NF), F32),
            pltpu.VMEM((BPW, NF), F32),
            pltpu.VMEM((BPW, 16), F32),
            pltpu.VMEM((BPW, 16), F32),
            pltpu.SemaphoreType.DMA,
        ],
    )


BLK = 1024
NBLK = BATCH // BLK


def _tc_math_body(th_ref, ps_ref, u_ref, i_ref, us_ref, is_ref, r_ref,
                  a_ref, b_ref, rhat_ref, ar_ref, loss_ref):
    p = th_ref[...]
    q = ps_ref[...]
    m = 0.5 * (p + q)
    lm = jnp.log(m)
    t = p * (jnp.log(p) - lm) + q * (jnp.log(q) - lm)
    asp = lax.broadcasted_iota(jnp.int32, (TW, NA), 0) // NT
    e = (asp == lax.broadcasted_iota(jnp.int32, (TW, NA), 1)).astype(F32)
    kl = jnp.dot(t, e, preferred_element_type=F32)
    s_uia = 1.0 - 0.5 * kl

    u = u_ref[...]
    i = i_ref[...]
    a = a_ref[...]
    ar = jnp.dot(u * i, (a * a).T, preferred_element_type=F32)
    a_hat = s_uia * ar
    ar_ref[...] = a_hat

    us = us_ref[...]
    isv = is_ref[...]
    pi = us[:, 0:1]
    bu = us[:, 1]
    lu = us[:, 2:10]
    bi = isv[:, 0]
    li = isv[:, 1:9]
    p_uia = pi * lu + (1.0 - pi) * li
    rhat = jnp.sum(p_uia * a_hat, axis=1) + bu + bi + b_ref[0]
    rhat_ref[...] = rhat

    res = r_ref[...] - rhat
    part = 0.5 * jnp.sum(res * res)
    part += 0.5 * L_U * jnp.sum(u * u)
    part += 0.5 * L_I * jnp.sum(i * i)
    part += 0.5 * L_B * (jnp.sum(bu * bu) + jnp.sum(bi * bi))

    @pl.when(pl.program_id(0) == 0)
    def _init():
        loss_ref[0] = 0.5 * L_A * jnp.sum(jnp.abs(a))

    loss_ref[0] += part


_tc_math = pl.pallas_call(
    _tc_math_body,
    grid=(NBLK,),
    in_specs=[
        pl.BlockSpec((BLK, TW), lambda b: (b, 0)),
        pl.BlockSpec((BLK, TW), lambda b: (b, 0)),
        pl.BlockSpec((BLK, NF), lambda b: (b, 0)),
        pl.BlockSpec((BLK, NF), lambda b: (b, 0)),
        pl.BlockSpec((BLK, 16), lambda b: (b, 0)),
        pl.BlockSpec((BLK, 16), lambda b: (b, 0)),
        pl.BlockSpec((BLK,), lambda b: (b,)),
        pl.BlockSpec((NA, NF), lambda b: (0, 0)),
        pl.BlockSpec((1,), lambda b: (0,)),
    ],
    out_specs=[
        pl.BlockSpec((BLK,), lambda b: (b,)),
        pl.BlockSpec((BLK, NA), lambda b: (b, 0)),
        pl.BlockSpec(memory_space=pltpu.SMEM),
    ],
    out_shape=[
        jax.ShapeDtypeStruct((BATCH,), F32),
        jax.ShapeDtypeStruct((BATCH, NA), F32),
        jax.ShapeDtypeStruct((1,), F32),
    ],
)


def kernel(U_ids, I_ids, R, user_table, item_table, Theta_u, Psi_i, Pi_u,
           Lambda_u, Lambda_i, A, Bu, Bi, B):
    th_flat = Theta_u.reshape(NU, TW)
    ps_flat = Psi_i.reshape(NI, TW)
    zu = jnp.zeros((NU, 6), F32)
    usm = jnp.concatenate([Pi_u[:, None], Bu[:, None], Lambda_u, zu], axis=1)
    zi = jnp.zeros((NI, 7), F32)
    ism = jnp.concatenate([Bi[:, None], Lambda_i, zi], axis=1)

    th_b, ps_b = _sc_theta()(U_ids, I_ids, th_flat, ps_flat)
    u_b, i_b, us_b, is_b = _sc_small()(
        U_ids, I_ids, user_table, item_table, usm, ism)

    rhat, a_hat, loss = _tc_math(th_b, ps_b, u_b, i_b, us_b, is_b, R, A, B)
    return rhat, a_hat, loss[0]
